# baseline (speedup 1.0000x reference)
import functools

import jax
import jax.numpy as jnp
from jax import lax
from jax.experimental import pallas as pl
from jax.experimental.pallas import tpu as pltpu
from jax.experimental.pallas import tpu_sc as plsc

DIM = 64
PAD = 128
NC, NS = 2, 16
NW = NC * NS

_MESH = plsc.VectorSubcoreMesh(core_axis_name="c", subcore_axis_name="s")
_PARAMS = pltpu.CompilerParams(use_tc_tiling_on_sc=True)

_CHUNK = 128


@functools.partial(jax.jit, static_argnames=("v",))
def _restage(emb, v):
    vd = v // 2
    n_main = vd // _CHUNK
    tail = vd - n_main * _CHUNK
    per_w = n_main // NW
    extra = n_main - per_w * NW

    @functools.partial(
        pl.kernel,
        out_type=jax.ShapeDtypeStruct((vd, PAD), jnp.float32),
        mesh=_MESH,
        compiler_params=_PARAMS,
        scratch_types=[
            pltpu.VMEM((2, 2 * _CHUNK, DIM), jnp.float32),
            pltpu.VMEM((2, _CHUNK, PAD), jnp.float32),
            pltpu.SemaphoreType.DMA,
            pltpu.SemaphoreType.DMA,
            pltpu.SemaphoreType.DMA,
            pltpu.SemaphoreType.DMA,
        ],
    )
    def k(emb_hbm, dense_hbm, a_v, b_v, ain0, ain1, bout0, bout1):
        wid = lax.axis_index("s") * NC + lax.axis_index("c")
        asems = (ain0, ain1)
        bsems = (bout0, bout1)

        def chunk_of(q):
            return wid + NW * q

        def fire_in(q, p):
            c = chunk_of(q)
            pltpu.async_copy(
                emb_hbm.at[pl.ds(c * 2 * _CHUNK, 2 * _CHUNK)], a_v.at[p], asems[p])

        def wait_in(p):
            pltpu.make_async_copy(
                emb_hbm.at[pl.ds(0, 2 * _CHUNK)], a_v.at[p], asems[p]).wait()

        def fire_out(q, p):
            c = chunk_of(q)
            pltpu.async_copy(
                b_v.at[p], dense_hbm.at[pl.ds(c * _CHUNK, _CHUNK)], bsems[p])

        def wait_out(p):
            pltpu.make_async_copy(
                b_v.at[p], dense_hbm.at[pl.ds(0, _CHUNK)], bsems[p]).wait()

        def repack(p):
            a = a_v.at[p]
            b = b_v.at[p]

            def row(j, carry):
                for c in range(4):
                    b[j, pl.ds(c * 16, 16)] = a[2 * j, pl.ds(c * 16, 16)]
                    b[j, pl.ds(DIM + c * 16, 16)] = a[2 * j + 1, pl.ds(c * 16, 16)]
                return carry

            lax.fori_loop(0, _CHUNK, row, 0)

        my_n = jnp.where(wid < extra, per_w + 1, per_w)

        fire_in(0, 0)

        @pl.when(my_n > 1)
        def _():
            fire_in(1, 1)

        def body(q, carry):
            p = lax.rem(q, 2)

            @pl.when(p == 0)
            def _():
                wait_in(0)
                @pl.when(q > 1)
                def _():
                    wait_out(0)
                repack(0)
                fire_out(q, 0)
                @pl.when(q + 2 < my_n)
                def _():
                    fire_in(q + 2, 0)

            @pl.when(p == 1)
            def _():
                wait_in(1)
                @pl.when(q > 1)
                def _():
                    wait_out(1)
                repack(1)
                fire_out(q, 1)
                @pl.when(q + 2 < my_n)
                def _():
                    fire_in(q + 2, 1)

            return carry

        lax.fori_loop(0, my_n, body, 0)
        wait_out(0)
        wait_out(1)

        @pl.when(wid == 2)
        def _():
            pltpu.sync_copy(
                emb_hbm.at[pl.ds(n_main * 2 * _CHUNK, 2 * tail)],
                a_v.at[0, pl.ds(0, 2 * tail)])

            def row(j, carry):
                for c in range(4):
                    b_v[0, j, pl.ds(c * 16, 16)] = a_v[0, 2 * j, pl.ds(c * 16, 16)]
                    b_v[0, j, pl.ds(DIM + c * 16, 16)] = (
                        a_v[0, 2 * j + 1, pl.ds(c * 16, 16)])
                return carry

            lax.fori_loop(0, tail, row, 0)
            pltpu.sync_copy(
                b_v.at[0, pl.ds(0, tail)],
                dense_hbm.at[pl.ds(n_main * _CHUNK, tail)])

    return k(emb)


@functools.partial(jax.jit, static_argnames=("b", "s"))
def _sc_gather(xf, dense, b, s):
    bpw = b // NW
    npos = bpw * s
    s0 = min(128, s)
    s1 = s - s0
    ng = (s + 15) // 16

    @functools.partial(
        pl.kernel,
        out_type=jax.ShapeDtypeStruct((b, s, DIM), jnp.float32),
        mesh=_MESH,
        compiler_params=_PARAMS,
        scratch_types=[
            pltpu.VMEM((npos + 32,), jnp.int32),
            pltpu.VMEM((2, 2, 128), jnp.int32),
            pltpu.VMEM((2, s, PAD), jnp.float32),
            pltpu.VMEM((2, s, DIM), jnp.float32),
            pltpu.SemaphoreType.DMA,
            pltpu.SemaphoreType.DMA,
            pltpu.SemaphoreType.DMA,
            pltpu.SemaphoreType.DMA,
        ],
    )
    def k(xf_hbm, dense_hbm, out_hbm, idx_all, sidx, g_v, c_v,
          gsem0, gsem1, osem0, osem1):
        wid = lax.axis_index("s") * NC + lax.axis_index("c")
        base = wid * bpw
        pltpu.sync_copy(xf_hbm.at[pl.ds(base * s, npos)], idx_all.at[pl.ds(0, npos)])
        zeros = jnp.zeros((16,), jnp.int32)
        idx_all[pl.ds(npos, 16)] = zeros
        idx_all[pl.ds(npos + 16, 16)] = zeros

        gsems = (gsem0, gsem1)
        osems = (osem0, osem1)

        def build_sidx(i, p):
            for g in range(ng):
                vals = idx_all[pl.ds(i * s + g * 16, 16)]
                half = jax.lax.shift_right_logical(vals, 1)
                if g < 8:
                    sidx[p, 0, pl.ds(g * 16, 16)] = half
                else:
                    sidx[p, 1, pl.ds((g - 8) * 16, 16)] = half

        def fire_gathers(p, sem):
            pltpu.async_copy(
                dense_hbm.at[sidx.at[p, 0]], g_v.at[p, pl.ds(0, s0)], sem)
            if s1:
                pltpu.async_copy(
                    dense_hbm.at[sidx.at[p, 1, pl.ds(0, s1)]],
                    g_v.at[p, pl.ds(s0, s1)], sem)

        def wait_gathers(p, sem):
            pltpu.make_async_copy(
                dense_hbm.at[pl.ds(0, s)], g_v.at[p], sem).wait()

        def compact(i, p):
            nfull = s // 16
            rem = s - nfull * 16

            def rows16(t, n):
                pvec = idx_all[pl.ds(i * s + t * 16, 16)]
                for e in range(n):
                    off = jax.lax.rem(pvec[e], 2) * DIM
                    j = t * 16 + e
                    for c in range(4):
                        c_v[p, j, pl.ds(c * 16, 16)] = (
                            g_v[p, j, pl.ds(off + c * 16, 16)])

            def grp(t, carry):
                rows16(t, 16)
                return carry

            lax.fori_loop(0, nfull, grp, 0)
            if rem:
                rows16(nfull, rem)

        def fire_store(i, p, sem):
            pltpu.async_copy(c_v.at[p], out_hbm.at[base + i], sem)

        def wait_store(p, sem):
            pltpu.make_async_copy(c_v.at[p], out_hbm.at[base], sem).wait()

        build_sidx(0, 0)
        fire_gathers(0, gsems[0])

        def body(i, carry):
            p = lax.rem(i, 2)

            @pl.when(p == 0)
            def _():
                @pl.when(i + 1 < bpw)
                def _():
                    build_sidx(i + 1, 1)
                    fire_gathers(1, gsems[1])
                wait_gathers(0, gsems[0])
                @pl.when(i > 1)
                def _():
                    wait_store(0, osems[0])
                compact(i, 0)
                fire_store(i, 0, osems[0])

            @pl.when(p == 1)
            def _():
                @pl.when(i + 1 < bpw)
                def _():
                    build_sidx(i + 1, 0)
                    fire_gathers(0, gsems[0])
                wait_gathers(1, gsems[1])
                @pl.when(i > 1)
                def _():
                    wait_store(1, osems[1])
                compact(i, 1)
                fire_store(i, 1, osems[1])

            return carry

        lax.fori_loop(0, bpw, body, 0)
        wait_store(0, osems[0])
        wait_store(1, osems[1])

    return k(xf, dense)


def kernel(x, emb):
    b, s = x.shape
    v = emb.shape[0]
    xf = x.astype(jnp.int32).reshape(b * s)
    dense = _restage(emb, v)
    return _sc_gather(xf, dense, b, s)


# speedup vs baseline: 1.1917x; 1.1917x over previous
import functools

import jax
import jax.numpy as jnp
from jax import lax
from jax.experimental import pallas as pl
from jax.experimental.pallas import tpu as pltpu
from jax.experimental.pallas import tpu_sc as plsc

DIM = 64
NC, NS = 2, 16
NW = NC * NS


@functools.partial(jax.jit, static_argnames=("b", "s"))
def _sc_gather(x, emb, b, s):
    bpw = b // NW
    n_half = bpw // 2
    s0 = min(128, s)
    s1 = s - s0
    mesh = plsc.VectorSubcoreMesh(core_axis_name="c", subcore_axis_name="s")

    @functools.partial(
        pl.kernel,
        out_type=jax.ShapeDtypeStruct((b, s, DIM), jnp.float32),
        mesh=mesh,
        compiler_params=pltpu.CompilerParams(
            use_tc_tiling_on_sc=False, needs_layout_passes=True),
        scratch_types=[
            pltpu.VMEM((bpw, s), jnp.int32),
            pltpu.VMEM((2, s, DIM), jnp.float32),
            pltpu.SemaphoreType.DMA,
            pltpu.SemaphoreType.DMA,
            pltpu.SemaphoreType.DMA,
            pltpu.SemaphoreType.DMA,
        ],
    )
    def k(x_hbm, emb_hbm, out_hbm, idx_all, rows_v, gsem0, gsem1, osem0, osem1):
        wid = lax.axis_index("s") * NC + lax.axis_index("c")
        base = wid * bpw
        pltpu.sync_copy(x_hbm.at[pl.ds(base, bpw)], idx_all)

        r0 = rows_v.at[0]
        r1 = rows_v.at[1]

        def fire_gathers(i, buf, sem):
            pltpu.async_copy(
                emb_hbm.at[idx_all.at[i, pl.ds(0, s0)]], buf.at[pl.ds(0, s0)], sem)
            if s1:
                pltpu.async_copy(
                    emb_hbm.at[idx_all.at[i, pl.ds(s0, s1)]], buf.at[pl.ds(s0, s1)], sem)

        def fire_store(i, buf, sem):
            pltpu.async_copy(buf, out_hbm.at[base + i], sem)

        def wait_bytes(buf, sem):
            pltpu.make_async_copy(buf, out_hbm.at[base], sem).wait()

        fire_gathers(0, r0, gsem0)

        def body2(t, carry):
            i0 = 2 * t

            @pl.when(t > 0)
            def _():
                wait_bytes(r1, osem1)
            fire_gathers(i0 + 1, r1, gsem1)
            wait_bytes(r0, gsem0)
            fire_store(i0, r0, osem0)

            @pl.when(t + 1 < n_half)
            def _():
                wait_bytes(r0, osem0)
                fire_gathers(i0 + 2, r0, gsem0)
            wait_bytes(r1, gsem1)
            fire_store(i0 + 1, r1, osem1)
            return carry

        lax.fori_loop(0, n_half, body2, 0)
        wait_bytes(r0, osem0)
        wait_bytes(r1, osem1)

    return k(x, emb)


def kernel(x, emb):
    b, s = x.shape
    return _sc_gather(x.astype(jnp.int32), emb, b, s)
